# baseline (speedup 1.0000x reference)
import functools

import jax
import jax.numpy as jnp
from jax import lax
from jax.experimental import pallas as pl
from jax.experimental.pallas import tpu as pltpu
from jax.experimental.pallas import tpu_sc as plsc

NUM_CORES = 2
NUM_SUBCORES = 16
NUM_WORKERS = NUM_CORES * NUM_SUBCORES
LANES = 16

ROWS_PER_GROUP = 4
NEUR_CHUNK = 128


def _build_sc_call(batch, input_size, num_neurons, num_synapses):
    assert batch % (NUM_WORKERS * ROWS_PER_GROUP) == 0
    assert num_neurons % (2 * NEUR_CHUNK) == 0 and NEUR_CHUNK % LANES == 0
    rows_per_worker = batch // NUM_WORKERS
    groups = rows_per_worker // ROWS_PER_GROUP
    chunks = num_neurons // NEUR_CHUNK
    nblocks = NEUR_CHUNK // LANES

    mesh = plsc.VectorSubcoreMesh(
        core_axis_name="c", subcore_axis_name="s", num_cores=NUM_CORES
    )

    @functools.partial(
        pl.kernel,
        out_type=jax.ShapeDtypeStruct((batch, num_synapses, num_neurons),
                                      jnp.float32),
        mesh=mesh,
        compiler_params=pltpu.CompilerParams(needs_layout_passes=False),
        scratch_types=[
            *[pltpu.VMEM((input_size,), jnp.float32) for _ in range(ROWS_PER_GROUP)],
            *[pltpu.VMEM((num_synapses, NEUR_CHUNK), jnp.int32) for _ in range(4)],
            *[pltpu.VMEM((ROWS_PER_GROUP, num_synapses, NEUR_CHUNK), jnp.float32)
              for _ in range(2)],
            pltpu.SemaphoreType.DMA,
            *[pltpu.SemaphoreType.DMA for _ in range(2)],
            *[pltpu.SemaphoreType.DMA for _ in range(2)],
        ],
    )
    def sc_call(x_hbm, ip_hbm, iq_hbm, out_hbm, r0, r1, r2, r3,
                ip0, iq0, ip1, iq1, ob0, ob1,
                row_sem, is0, is1, os0, os1):
        rows = [r0, r1, r2, r3]
        idx_bufs = [(ip0, iq0), (ip1, iq1)]
        idx_sems = [is0, is1]
        out_bufs = [ob0, ob1]
        out_sems = [os0, os1]
        wid = lax.axis_index("s") * NUM_CORES + lax.axis_index("c")
        base = wid * rows_per_worker

        def idx_slice(c):
            return pl.ds(c * NEUR_CHUNK, NEUR_CHUNK)

        def start_idx(c, buf):
            pltpu.async_copy(ip_hbm.at[:, idx_slice(c)], idx_bufs[buf][0],
                             idx_sems[buf])
            pltpu.async_copy(iq_hbm.at[:, idx_slice(c)], idx_bufs[buf][1],
                             idx_sems[buf])

        def wait_idx(c, buf):
            pltpu.make_async_copy(ip_hbm.at[:, idx_slice(c)],
                                  idx_bufs[buf][0], idx_sems[buf]).wait()
            pltpu.make_async_copy(iq_hbm.at[:, idx_slice(c)],
                                  idx_bufs[buf][1], idx_sems[buf]).wait()

        def out_slice(row0, c):
            return out_hbm.at[pl.ds(row0, ROWS_PER_GROUP), :,
                              pl.ds(c * NEUR_CHUNK, NEUR_CHUNK)]

        def start_out(row0, c, buf):
            pltpu.async_copy(out_bufs[buf], out_slice(row0, c), out_sems[buf])

        def wait_out(row0, c, buf):
            pltpu.make_async_copy(out_bufs[buf], out_slice(row0, c),
                                  out_sems[buf]).wait()

        def compute_chunk(buf):
            ipv, iqv = idx_bufs[buf]
            ob = out_bufs[buf]

            @plsc.parallel_loop(0, num_synapses, 1, unroll=4)
            def vec_body(s):
                for nb in range(nblocks):
                    off = nb * LANES
                    ip = ipv[s, pl.ds(off, LANES)]
                    iq = iqv[s, pl.ds(off, LANES)]
                    for r in range(ROWS_PER_GROUP):
                        u = plsc.load_gather(rows[r], [ip])
                        v = plsc.load_gather(rows[r], [iq])
                        ob[r, s, pl.ds(off, LANES)] = jnp.where(
                            u > v, jnp.float32(1.0), jnp.float32(0.0)
                        )

        def group_body(g, carry):
            row0 = base + g * ROWS_PER_GROUP
            for r in range(ROWS_PER_GROUP):
                pltpu.async_copy(x_hbm.at[row0 + r], rows[r], row_sem)
            for r in range(ROWS_PER_GROUP):
                pltpu.make_async_copy(x_hbm.at[row0 + r], rows[r],
                                      row_sem).wait()
            start_idx(0, 0)
            start_idx(1, 1)

            def pair_body(c2, carry):
                c = c2 * 2
                for buf in range(2):
                    wait_idx(c + buf, buf)

                    @pl.when(c2 > 0)
                    def _():
                        wait_out(row0, c + buf - 2, buf)

                    compute_chunk(buf)
                    start_out(row0, c + buf, buf)

                    @pl.when(c2 < chunks // 2 - 1)
                    def _():
                        start_idx(c + buf + 2, buf)

                return carry

            lax.fori_loop(0, chunks // 2, pair_body, carry, unroll=False)
            for buf in range(2):
                wait_out(row0, chunks - 2 + buf, buf)
            return carry

        lax.fori_loop(0, groups, group_body, 0, unroll=False)

    return sc_call


def kernel(x, idx_p, idx_q):
    batch, input_size = x.shape
    num_neurons, num_synapses = idx_p.shape
    ip = idx_p.T.astype(jnp.int32)
    iq = idx_q.T.astype(jnp.int32)
    sc_call = _build_sc_call(batch, input_size, num_neurons, num_synapses)
    out_bsn = sc_call(x, ip, iq)
    return out_bsn.transpose(0, 2, 1)


# speedup vs baseline: 1.1873x; 1.1873x over previous
import functools

import jax
import jax.numpy as jnp
from jax import lax
from jax.experimental import pallas as pl
from jax.experimental.pallas import tpu as pltpu
from jax.experimental.pallas import tpu_sc as plsc

NUM_CORES = 2
NUM_SUBCORES = 16
NUM_WORKERS = NUM_CORES * NUM_SUBCORES
LANES = 16

ROWS_PER_GROUP = 4
NEUR_CHUNK = 128


def _build_sc_call(batch, input_size, num_neurons, num_synapses):
    assert batch % (NUM_WORKERS * ROWS_PER_GROUP) == 0
    assert num_neurons % (2 * NEUR_CHUNK) == 0 and NEUR_CHUNK % LANES == 0
    rows_per_worker = batch // NUM_WORKERS
    groups = rows_per_worker // ROWS_PER_GROUP
    chunks = num_neurons // NEUR_CHUNK
    nblocks = NEUR_CHUNK // LANES

    mesh = plsc.VectorSubcoreMesh(
        core_axis_name="c", subcore_axis_name="s", num_cores=NUM_CORES
    )

    @functools.partial(
        pl.kernel,
        out_type=jax.ShapeDtypeStruct((batch, num_synapses, num_neurons),
                                      jnp.float32),
        mesh=mesh,
        compiler_params=pltpu.CompilerParams(needs_layout_passes=False),
        scratch_types=[
            *[pltpu.VMEM((input_size,), jnp.float32) for _ in range(ROWS_PER_GROUP)],
            *[pltpu.VMEM((num_synapses, NEUR_CHUNK), jnp.int32) for _ in range(4)],
            *[pltpu.VMEM((ROWS_PER_GROUP, num_synapses, NEUR_CHUNK), jnp.float32)
              for _ in range(2)],
            pltpu.SemaphoreType.DMA,
            *[pltpu.SemaphoreType.DMA for _ in range(2)],
            *[pltpu.SemaphoreType.DMA for _ in range(2)],
        ],
    )
    def sc_call(x_hbm, ip_hbm, iq_hbm, out_hbm, r0, r1, r2, r3,
                ip0, iq0, ip1, iq1, ob0, ob1,
                row_sem, is0, is1, os0, os1):
        rows = [r0, r1, r2, r3]
        idx_bufs = [(ip0, iq0), (ip1, iq1)]
        idx_sems = [is0, is1]
        out_bufs = [ob0, ob1]
        out_sems = [os0, os1]
        wid = lax.axis_index("s") * NUM_CORES + lax.axis_index("c")
        base = wid * rows_per_worker

        def idx_slice(c):
            return pl.ds(c * NEUR_CHUNK, NEUR_CHUNK)

        def start_idx(c, buf):
            pltpu.async_copy(ip_hbm.at[:, idx_slice(c)], idx_bufs[buf][0],
                             idx_sems[buf])
            pltpu.async_copy(iq_hbm.at[:, idx_slice(c)], idx_bufs[buf][1],
                             idx_sems[buf])

        def wait_idx(c, buf):
            pltpu.make_async_copy(ip_hbm.at[:, idx_slice(c)],
                                  idx_bufs[buf][0], idx_sems[buf]).wait()
            pltpu.make_async_copy(iq_hbm.at[:, idx_slice(c)],
                                  idx_bufs[buf][1], idx_sems[buf]).wait()

        def out_slice(row0, c):
            return out_hbm.at[pl.ds(row0, ROWS_PER_GROUP), :,
                              pl.ds(c * NEUR_CHUNK, NEUR_CHUNK)]

        def start_out(row0, c, buf):
            pltpu.async_copy(out_bufs[buf], out_slice(row0, c), out_sems[buf])

        def wait_out(row0, c, buf):
            pltpu.make_async_copy(out_bufs[buf], out_slice(row0, c),
                                  out_sems[buf]).wait()

        def compute_chunk(buf):
            ipv, iqv = idx_bufs[buf]
            ob = out_bufs[buf]

            @plsc.parallel_loop(0, num_synapses, 1, unroll=1)
            def vec_body(s):
                for nb in range(nblocks):
                    off = nb * LANES
                    ip = ipv[s, pl.ds(off, LANES)]
                    iq = iqv[s, pl.ds(off, LANES)]
                    for r in range(ROWS_PER_GROUP):
                        u = plsc.load_gather(rows[r], [ip])
                        v = plsc.load_gather(rows[r], [iq])
                        ob[r, s, pl.ds(off, LANES)] = jnp.where(
                            u > v, jnp.float32(1.0), jnp.float32(0.0)
                        )

        def group_body(g, carry):
            row0 = base + g * ROWS_PER_GROUP
            for r in range(ROWS_PER_GROUP):
                pltpu.async_copy(x_hbm.at[row0 + r], rows[r], row_sem)
            for r in range(ROWS_PER_GROUP):
                pltpu.make_async_copy(x_hbm.at[row0 + r], rows[r],
                                      row_sem).wait()
            start_idx(0, 0)
            start_idx(1, 1)

            def pair_body(c2, carry):
                c = c2 * 2
                for buf in range(2):
                    wait_idx(c + buf, buf)

                    @pl.when(c2 > 0)
                    def _():
                        wait_out(row0, c + buf - 2, buf)

                    compute_chunk(buf)
                    start_out(row0, c + buf, buf)

                    @pl.when(c2 < chunks // 2 - 1)
                    def _():
                        start_idx(c + buf + 2, buf)

                return carry

            lax.fori_loop(0, chunks // 2, pair_body, carry, unroll=False)
            for buf in range(2):
                wait_out(row0, chunks - 2 + buf, buf)
            return carry

        lax.fori_loop(0, groups, group_body, 0, unroll=False)

    return sc_call


def kernel(x, idx_p, idx_q):
    batch, input_size = x.shape
    num_neurons, num_synapses = idx_p.shape
    ip = idx_p.T.astype(jnp.int32)
    iq = idx_q.T.astype(jnp.int32)
    sc_call = _build_sc_call(batch, input_size, num_neurons, num_synapses)
    out_bsn = sc_call(x, ip, iq)
    return out_bsn.transpose(0, 2, 1)


# speedup vs baseline: 1.2604x; 1.0615x over previous
import functools

import jax
import jax.numpy as jnp
from jax import lax
from jax.experimental import pallas as pl
from jax.experimental.pallas import tpu as pltpu
from jax.experimental.pallas import tpu_sc as plsc

NUM_CORES = 2
NUM_SUBCORES = 16
NUM_WORKERS = NUM_CORES * NUM_SUBCORES
LANES = 16

ROWS_PER_GROUP = 4
NEUR_CHUNK = 128


def _build_sc_call(batch, input_size, num_neurons, num_synapses):
    assert batch % (NUM_WORKERS * ROWS_PER_GROUP) == 0
    assert num_neurons % (2 * NEUR_CHUNK) == 0 and NEUR_CHUNK % LANES == 0
    rows_per_worker = batch // NUM_WORKERS
    groups = rows_per_worker // ROWS_PER_GROUP
    chunks = num_neurons // NEUR_CHUNK
    nblocks = NEUR_CHUNK // LANES

    mesh = plsc.VectorSubcoreMesh(
        core_axis_name="c", subcore_axis_name="s", num_cores=NUM_CORES
    )

    @functools.partial(
        pl.kernel,
        out_type=jax.ShapeDtypeStruct((batch, num_synapses, num_neurons),
                                      jnp.float32),
        mesh=mesh,
        compiler_params=pltpu.CompilerParams(needs_layout_passes=False),
        scratch_types=[
            *[pltpu.VMEM((input_size,), jnp.float32) for _ in range(ROWS_PER_GROUP)],
            *[pltpu.VMEM((num_synapses, NEUR_CHUNK), jnp.int32) for _ in range(4)],
            *[pltpu.VMEM((ROWS_PER_GROUP, num_synapses, NEUR_CHUNK), jnp.float32)
              for _ in range(2)],
            *[pltpu.VMEM_SHARED((num_synapses, num_neurons), jnp.int32)
              for _ in range(2)],
            pltpu.SemaphoreType.DMA,
            *[pltpu.SemaphoreType.DMA for _ in range(2)],
            *[pltpu.SemaphoreType.DMA for _ in range(2)],
        ],
    )
    def sc_call(x_hbm, ip_hbm, iq_hbm, out_hbm, r0, r1, r2, r3,
                ip0, iq0, ip1, iq1, ob0, ob1, ip_sp, iq_sp,
                row_sem, is0, is1, os0, os1):
        rows = [r0, r1, r2, r3]
        idx_bufs = [(ip0, iq0), (ip1, iq1)]
        idx_sems = [is0, is1]
        out_bufs = [ob0, ob1]
        out_sems = [os0, os1]
        wid = lax.axis_index("s") * NUM_CORES + lax.axis_index("c")
        base = wid * rows_per_worker

        def idx_slice(c):
            return pl.ds(c * NEUR_CHUNK, NEUR_CHUNK)

        def start_idx(c, buf):
            pltpu.async_copy(ip_sp.at[:, idx_slice(c)], idx_bufs[buf][0],
                             idx_sems[buf])
            pltpu.async_copy(iq_sp.at[:, idx_slice(c)], idx_bufs[buf][1],
                             idx_sems[buf])

        def wait_idx(c, buf):
            pltpu.make_async_copy(ip_sp.at[:, idx_slice(c)],
                                  idx_bufs[buf][0], idx_sems[buf]).wait()
            pltpu.make_async_copy(iq_sp.at[:, idx_slice(c)],
                                  idx_bufs[buf][1], idx_sems[buf]).wait()

        def out_slice(row0, c):
            return out_hbm.at[pl.ds(row0, ROWS_PER_GROUP), :,
                              pl.ds(c * NEUR_CHUNK, NEUR_CHUNK)]

        def start_out(row0, c, buf):
            pltpu.async_copy(out_bufs[buf], out_slice(row0, c), out_sems[buf])

        def wait_out(row0, c, buf):
            pltpu.make_async_copy(out_bufs[buf], out_slice(row0, c),
                                  out_sems[buf]).wait()

        def compute_chunk(buf):
            ipv, iqv = idx_bufs[buf]
            ob = out_bufs[buf]

            @plsc.parallel_loop(0, num_synapses, 1, unroll=1)
            def vec_body(s):
                for nb in range(nblocks):
                    off = nb * LANES
                    ip = ipv[s, pl.ds(off, LANES)]
                    iq = iqv[s, pl.ds(off, LANES)]
                    for r in range(ROWS_PER_GROUP):
                        u = plsc.load_gather(rows[r], [ip])
                        v = plsc.load_gather(rows[r], [iq])
                        ob[r, s, pl.ds(off, LANES)] = jnp.where(
                            u > v, jnp.float32(1.0), jnp.float32(0.0)
                        )

        @pl.when(lax.axis_index("s") == 0)
        def _():
            pltpu.sync_copy(ip_hbm, ip_sp)
            pltpu.sync_copy(iq_hbm, iq_sp)

        plsc.subcore_barrier()

        def group_body(g, carry):
            row0 = base + g * ROWS_PER_GROUP
            for r in range(ROWS_PER_GROUP):
                pltpu.async_copy(x_hbm.at[row0 + r], rows[r], row_sem)
            for r in range(ROWS_PER_GROUP):
                pltpu.make_async_copy(x_hbm.at[row0 + r], rows[r],
                                      row_sem).wait()
            start_idx(0, 0)
            start_idx(1, 1)

            def pair_body(c2, carry):
                c = c2 * 2
                for buf in range(2):
                    wait_idx(c + buf, buf)

                    @pl.when(c2 > 0)
                    def _():
                        wait_out(row0, c + buf - 2, buf)

                    compute_chunk(buf)
                    start_out(row0, c + buf, buf)

                    @pl.when(c2 < chunks // 2 - 1)
                    def _():
                        start_idx(c + buf + 2, buf)

                return carry

            lax.fori_loop(0, chunks // 2, pair_body, carry, unroll=False)
            for buf in range(2):
                wait_out(row0, chunks - 2 + buf, buf)
            return carry

        lax.fori_loop(0, groups, group_body, 0, unroll=False)

    return sc_call


def kernel(x, idx_p, idx_q):
    batch, input_size = x.shape
    num_neurons, num_synapses = idx_p.shape
    ip = idx_p.T.astype(jnp.int32)
    iq = idx_q.T.astype(jnp.int32)
    sc_call = _build_sc_call(batch, input_size, num_neurons, num_synapses)
    out_bsn = sc_call(x, ip, iq)
    return out_bsn.transpose(0, 2, 1)


# speedup vs baseline: 1.2845x; 1.0191x over previous
import functools

import jax
import jax.numpy as jnp
from jax import lax
from jax.experimental import pallas as pl
from jax.experimental.pallas import tpu as pltpu
from jax.experimental.pallas import tpu_sc as plsc

NUM_CORES = 2
NUM_SUBCORES = 16
NUM_WORKERS = NUM_CORES * NUM_SUBCORES
LANES = 16

ROWS_PER_GROUP = 4
NEUR_CHUNK = 128


def _build_sc_call(batch, input_size, num_neurons, num_synapses):
    assert batch % (NUM_WORKERS * ROWS_PER_GROUP) == 0
    assert num_neurons % (2 * NEUR_CHUNK) == 0 and NEUR_CHUNK % LANES == 0
    rows_per_worker = batch // NUM_WORKERS
    groups = rows_per_worker // ROWS_PER_GROUP
    chunks = num_neurons // NEUR_CHUNK
    nblocks = NEUR_CHUNK // LANES

    mesh = plsc.VectorSubcoreMesh(
        core_axis_name="c", subcore_axis_name="s", num_cores=NUM_CORES
    )

    @functools.partial(
        pl.kernel,
        out_type=jax.ShapeDtypeStruct((batch, num_synapses, num_neurons),
                                      jnp.float32),
        mesh=mesh,
        compiler_params=pltpu.CompilerParams(needs_layout_passes=False),
        scratch_types=[
            *[pltpu.VMEM((input_size,), jnp.float32) for _ in range(ROWS_PER_GROUP)],
            *[pltpu.VMEM((num_synapses, NEUR_CHUNK), jnp.int32) for _ in range(4)],
            *[pltpu.VMEM((ROWS_PER_GROUP, num_synapses, NEUR_CHUNK), jnp.float32)
              for _ in range(2)],
            *[pltpu.VMEM_SHARED((num_synapses, num_neurons), jnp.int32)
              for _ in range(2)],
            pltpu.SemaphoreType.DMA,
            *[pltpu.SemaphoreType.DMA for _ in range(2)],
            *[pltpu.SemaphoreType.DMA for _ in range(2)],
        ],
    )
    def sc_call(x_hbm, ip_hbm, iq_hbm, out_hbm, r0, r1, r2, r3,
                ip0, iq0, ip1, iq1, ob0, ob1, ip_sp, iq_sp,
                row_sem, is0, is1, os0, os1):
        rows = [r0, r1, r2, r3]
        idx_bufs = [(ip0, iq0), (ip1, iq1)]
        idx_sems = [is0, is1]
        out_bufs = [ob0, ob1]
        out_sems = [os0, os1]
        wid = lax.axis_index("s") * NUM_CORES + lax.axis_index("c")
        base = wid * rows_per_worker

        def idx_slice(c):
            return pl.ds(c * NEUR_CHUNK, NEUR_CHUNK)

        def start_idx(c, buf):
            pltpu.async_copy(ip_sp.at[:, idx_slice(c)], idx_bufs[buf][0],
                             idx_sems[buf])
            pltpu.async_copy(iq_sp.at[:, idx_slice(c)], idx_bufs[buf][1],
                             idx_sems[buf])

        def wait_idx(c, buf):
            pltpu.make_async_copy(ip_sp.at[:, idx_slice(c)],
                                  idx_bufs[buf][0], idx_sems[buf]).wait()
            pltpu.make_async_copy(iq_sp.at[:, idx_slice(c)],
                                  idx_bufs[buf][1], idx_sems[buf]).wait()

        def out_slice(row0, c):
            return out_hbm.at[pl.ds(row0, ROWS_PER_GROUP), :,
                              pl.ds(c * NEUR_CHUNK, NEUR_CHUNK)]

        def start_out(row0, c, buf):
            pltpu.async_copy(out_bufs[buf], out_slice(row0, c), out_sems[buf])

        def wait_out(row0, c, buf):
            pltpu.make_async_copy(out_bufs[buf], out_slice(row0, c),
                                  out_sems[buf]).wait()

        def compute_chunk(buf):
            ipv, iqv = idx_bufs[buf]
            ob = out_bufs[buf]

            @plsc.parallel_loop(0, num_synapses, 1, unroll=1)
            def vec_body(s):
                for nb in range(nblocks):
                    off = nb * LANES
                    ip = ipv[s, pl.ds(off, LANES)]
                    iq = iqv[s, pl.ds(off, LANES)]
                    for r in range(ROWS_PER_GROUP):
                        u = plsc.load_gather(rows[r], [ip])
                        v = plsc.load_gather(rows[r], [iq])
                        ob[r, s, pl.ds(off, LANES)] = jnp.where(
                            u > v, jnp.float32(1.0), jnp.float32(0.0)
                        )

        @pl.when(lax.axis_index("s") == 0)
        def _():
            pltpu.sync_copy(ip_hbm, ip_sp)
            pltpu.sync_copy(iq_hbm, iq_sp)

        plsc.subcore_barrier()

        def start_rows(row0):
            for r in range(ROWS_PER_GROUP):
                pltpu.async_copy(x_hbm.at[row0 + r], rows[r], row_sem)

        def wait_rows(row0):
            for r in range(ROWS_PER_GROUP):
                pltpu.make_async_copy(x_hbm.at[row0 + r], rows[r],
                                      row_sem).wait()

        start_rows(base)
        start_idx(0, 0)
        start_idx(1, 1)

        def group_body(g, carry):
            row0 = base + g * ROWS_PER_GROUP
            wait_rows(row0)

            def pair_body(c2, carry):
                c = c2 * 2
                for buf in range(2):
                    wait_idx(c + buf, buf)

                    @pl.when(c2 > 0)
                    def _():
                        wait_out(row0, c + buf - 2, buf)

                    compute_chunk(buf)
                    start_out(row0, c + buf, buf)

                    @pl.when(c2 < chunks // 2 - 1)
                    def _():
                        start_idx(c + buf + 2, buf)

                return carry

            lax.fori_loop(0, chunks // 2, pair_body, carry, unroll=False)

            @pl.when(g < groups - 1)
            def _():
                start_rows(row0 + ROWS_PER_GROUP)
                start_idx(0, 0)
                start_idx(1, 1)

            for buf in range(2):
                wait_out(row0, chunks - 2 + buf, buf)
            return carry

        lax.fori_loop(0, groups, group_body, 0, unroll=False)

    return sc_call


def kernel(x, idx_p, idx_q):
    batch, input_size = x.shape
    num_neurons, num_synapses = idx_p.shape
    ip = idx_p.T.astype(jnp.int32)
    iq = idx_q.T.astype(jnp.int32)
    sc_call = _build_sc_call(batch, input_size, num_neurons, num_synapses)
    out_bsn = sc_call(x, ip, iq)
    return out_bsn.transpose(0, 2, 1)
